# baseline (speedup 1.0000x reference)
import functools

import jax
import jax.numpy as jnp
from jax import lax
from jax.experimental import pallas as pl
from jax.experimental.pallas import tpu as pltpu
from jax.experimental.pallas import tpu_sc as plsc

NUM_CORES = 2
NUM_SUBCORES = 16
NUM_WORKERS = NUM_CORES * NUM_SUBCORES
LANES = 16

SEQS_PER_CHUNK = 2
DETILE_ROWS = 1024
RETILE_BATCH = 2

_MESH = plsc.VectorSubcoreMesh(core_axis_name="c", subcore_axis_name="s")


def _make_lookup(B, S, D, V):
    assert B % NUM_WORKERS == 0
    seqs_per_worker = B // NUM_WORKERS
    assert seqs_per_worker % (2 * SEQS_PER_CHUNK) == 0
    chunks = seqs_per_worker // SEQS_PER_CHUNK
    toks = SEQS_PER_CHUNK * S
    assert D == 2 * LANES

    sub = []
    for q in range(SEQS_PER_CHUNK):
        off = 0
        while off < S:
            sz = min(128, S - off)
            sub.append((q, off, sz))
            off += sz

    @functools.partial(
        pl.kernel,
        mesh=_MESH,
        compiler_params=pltpu.CompilerParams(use_tc_tiling_on_sc=False),
        out_type=jax.ShapeDtypeStruct((B, S, D), jnp.float32),
        scratch_types=[
            pltpu.VMEM((SEQS_PER_CHUNK, S), jnp.int32),
            pltpu.VMEM((SEQS_PER_CHUNK, S), jnp.int32),
            pltpu.VMEM((SEQS_PER_CHUNK, S, D), jnp.float32),
            pltpu.VMEM((SEQS_PER_CHUNK, S, D), jnp.float32),
            pltpu.VMEM((S, D), jnp.float32),
            pltpu.SemaphoreType.DMA,
            pltpu.SemaphoreType.DMA,
            pltpu.SemaphoreType.DMA,
            pltpu.SemaphoreType.DMA,
        ],
    )
    def lookup(table_hbm, idx_hbm, pos_hbm, out_hbm,
               idx0_v, idx1_v, rows0_v, rows1_v, pos_v,
               gsem0, gsem1, osem0, osem1):
        wid = lax.axis_index("s") * NUM_CORES + lax.axis_index("c")
        seq_base = wid * seqs_per_worker

        idx_v = (idx0_v, idx1_v)
        rows_v = (rows0_v, rows1_v)
        gsem = (gsem0, gsem1)
        osem = (osem0, osem1)

        pltpu.sync_copy(pos_hbm, pos_v)

        def gather_copies(buf):
            cps = []
            for (q, r, sz) in sub:
                cps.append(
                    pltpu.make_async_copy(
                        table_hbm.at[idx_v[buf].at[q, pl.ds(r, sz)]],
                        rows_v[buf].at[q, pl.ds(r, sz)],
                        gsem[buf],
                    )
                )
            return cps

        def start_chunk(c, buf):
            b0 = pl.multiple_of(seq_base + c * SEQS_PER_CHUNK, SEQS_PER_CHUNK)
            pltpu.sync_copy(idx_hbm.at[pl.ds(b0, SEQS_PER_CHUNK)], idx_v[buf])
            for cp in gather_copies(buf):
                cp.start()

        def out_copy(c, buf):
            b0 = pl.multiple_of(seq_base + c * SEQS_PER_CHUNK, SEQS_PER_CHUNK)
            return pltpu.make_async_copy(
                rows_v[buf], out_hbm.at[pl.ds(b0, SEQS_PER_CHUNK)], osem[buf]
            )

        def add_pos(buf):
            def add_body(s, carry2):
                p0 = pos_v[s, pl.ds(0, LANES)]
                p1 = pos_v[s, pl.ds(LANES, LANES)]
                for q in range(SEQS_PER_CHUNK):
                    rows_v[buf][q, s, pl.ds(0, LANES)] += p0
                    rows_v[buf][q, s, pl.ds(LANES, LANES)] += p1
                return carry2

            lax.fori_loop(0, S, add_body, 0)

        start_chunk(0, 0)

        def step(i, carry):
            for buf in (0, 1):
                c = i * 2 + buf
                for cp in gather_copies(buf):
                    cp.wait()

                @pl.when(c >= 1)
                def _():
                    out_copy(c - 1, 1 - buf).wait()

                @pl.when(c + 1 < chunks)
                def _():
                    start_chunk(c + 1, 1 - buf)

                add_pos(buf)
                out_copy(c, buf).start()
            return carry

        lax.fori_loop(0, chunks // 2, step, 0)
        out_copy(chunks - 1, 1).wait()

    return lookup


def _make_retile(B, S, D):
    assert B % (NUM_WORKERS * RETILE_BATCH * 2) == 0
    windows = B // (NUM_WORKERS * RETILE_BATCH)
    wsz = RETILE_BATCH * S * D

    @functools.partial(
        pl.kernel,
        mesh=_MESH,
        compiler_params=pltpu.CompilerParams(use_tc_tiling_on_sc=True),
        out_type=jax.ShapeDtypeStruct((B, S, D), jnp.float32),
        scratch_types=[
            pltpu.VMEM((wsz,), jnp.float32),
            pltpu.VMEM((wsz,), jnp.float32),
            pltpu.VMEM((RETILE_BATCH, S, D), jnp.float32),
            pltpu.VMEM((RETILE_BATCH, S, D), jnp.float32),
            pltpu.SemaphoreType.DMA,
            pltpu.SemaphoreType.DMA,
            pltpu.SemaphoreType.DMA,
            pltpu.SemaphoreType.DMA,
        ],
    )
    def retile(flat_hbm, out_hbm, fin0, fin1, bo0, bo1, isem0, isem1,
               osem0, osem1):
        wid = lax.axis_index("s") * NUM_CORES + lax.axis_index("c")
        fin = (fin0, fin1)
        bo = (bo0, bo1)
        isem = (isem0, isem1)
        osem = (osem0, osem1)
        base = wid * windows * RETILE_BATCH

        def in_copy(j, p):
            o0 = pl.multiple_of((base + j * RETILE_BATCH) * S * D, 8)
            return pltpu.make_async_copy(
                flat_hbm.at[pl.ds(o0, wsz)], fin[p], isem[p]
            )

        def out_copy(j, p):
            b0 = pl.multiple_of(base + j * RETILE_BATCH, RETILE_BATCH)
            return pltpu.make_async_copy(
                bo[p], out_hbm.at[pl.ds(b0, RETILE_BATCH)], osem[p]
            )

        def shuffle(p):
            def body(si, carry):
                for q in range(RETILE_BATCH):
                    o = (q * S + si) * D
                    bo[p][q, si, pl.ds(0, LANES)] = fin[p][pl.ds(o, LANES)]
                    bo[p][q, si, pl.ds(LANES, LANES)] = (
                        fin[p][pl.ds(o + LANES, LANES)]
                    )
                return carry

            lax.fori_loop(0, S, body, 0)

        in_copy(0, 0).start()

        def step(i, carry):
            for p in (0, 1):
                j = i * 2 + p
                in_copy(j, p).wait()

                @pl.when(j + 1 < windows)
                def _():
                    in_copy(j + 1, 1 - p).start()

                @pl.when(j >= 2)
                def _():
                    out_copy(j - 2, p).wait()

                shuffle(p)
                out_copy(j, p).start()
            return carry

        lax.fori_loop(0, windows // 2, step, 0)
        out_copy(windows - 2, 0).wait()
        out_copy(windows - 1, 1).wait()

    return retile


def kernel(indices, token_table, pos_table):
    B, S = indices.shape
    V, D = token_table.shape
    pos_rows = lax.slice(pos_table, (0, 0), (S, D))
    out3 = _make_lookup(B, S, D, V)(
        token_table, indices.astype(jnp.int32), pos_rows
    )
    return _make_retile(B, S, D)(out3.reshape(B * S * D))


# speedup vs baseline: 1.0361x; 1.0361x over previous
import functools

import jax
import jax.numpy as jnp
from jax import lax
from jax.experimental import pallas as pl
from jax.experimental.pallas import tpu as pltpu
from jax.experimental.pallas import tpu_sc as plsc

NUM_CORES = 2
NUM_SUBCORES = 16
NUM_WORKERS = NUM_CORES * NUM_SUBCORES
LANES = 16

SEQS_PER_CHUNK = 2


def _make_lookup(B, S, D):
    assert B % NUM_WORKERS == 0
    seqs_per_worker = B // NUM_WORKERS
    assert seqs_per_worker % (2 * SEQS_PER_CHUNK) == 0
    chunks = seqs_per_worker // SEQS_PER_CHUNK
    toks = SEQS_PER_CHUNK * S
    assert D == 2 * LANES

    sub = []
    for q in range(SEQS_PER_CHUNK):
        off = 0
        while off < S:
            sz = min(128, S - off)
            sub.append((q, off, sz))
            off += sz

    mesh = plsc.VectorSubcoreMesh(core_axis_name="c", subcore_axis_name="s")

    @functools.partial(
        pl.kernel,
        mesh=mesh,
        compiler_params=pltpu.CompilerParams(use_tc_tiling_on_sc=False),
        out_type=jax.ShapeDtypeStruct((B, S, D), jnp.float32),
        scratch_types=[
            pltpu.VMEM((SEQS_PER_CHUNK, S), jnp.int32),
            pltpu.VMEM((SEQS_PER_CHUNK, S), jnp.int32),
            pltpu.VMEM((SEQS_PER_CHUNK, S, D), jnp.float32),
            pltpu.VMEM((SEQS_PER_CHUNK, S, D), jnp.float32),
            pltpu.VMEM((S, D), jnp.float32),
            pltpu.SemaphoreType.DMA,
            pltpu.SemaphoreType.DMA,
            pltpu.SemaphoreType.DMA,
            pltpu.SemaphoreType.DMA,
        ],
    )
    def lookup(table_hbm, idx_hbm, pos_hbm, out_hbm,
               idx0_v, idx1_v, rows0_v, rows1_v, pos_v,
               gsem0, gsem1, osem0, osem1):
        wid = lax.axis_index("s") * NUM_CORES + lax.axis_index("c")
        seq_base = wid * seqs_per_worker

        idx_v = (idx0_v, idx1_v)
        rows_v = (rows0_v, rows1_v)
        gsem = (gsem0, gsem1)
        osem = (osem0, osem1)

        pltpu.sync_copy(pos_hbm, pos_v)

        def gather_copies(c, buf):
            b0 = pl.multiple_of(seq_base + c * SEQS_PER_CHUNK, SEQS_PER_CHUNK)
            cps = []
            for (q, r, sz) in sub:
                cps.append(
                    pltpu.make_async_copy(
                        table_hbm.at[idx_v[buf].at[q, pl.ds(r, sz)]],
                        rows_v[buf].at[q, pl.ds(r, sz)],
                        gsem[buf],
                    )
                )
            return b0, cps

        def start_chunk(c, buf):
            b0 = pl.multiple_of(seq_base + c * SEQS_PER_CHUNK, SEQS_PER_CHUNK)
            pltpu.sync_copy(idx_hbm.at[pl.ds(b0, SEQS_PER_CHUNK)], idx_v[buf])
            _, cps = gather_copies(c, buf)
            for cp in cps:
                cp.start()

        def out_copy(c, buf):
            b0 = pl.multiple_of(seq_base + c * SEQS_PER_CHUNK, SEQS_PER_CHUNK)
            return pltpu.make_async_copy(
                rows_v[buf], out_hbm.at[pl.ds(b0, SEQS_PER_CHUNK)], osem[buf]
            )

        def add_pos(buf):
            def add_body(s, carry2):
                p0 = pos_v[s, pl.ds(0, LANES)]
                p1 = pos_v[s, pl.ds(LANES, LANES)]
                for q in range(SEQS_PER_CHUNK):
                    rows_v[buf][q, s, pl.ds(0, LANES)] += p0
                    rows_v[buf][q, s, pl.ds(LANES, LANES)] += p1
                return carry2

            lax.fori_loop(0, S, add_body, 0)

        start_chunk(0, 0)

        def step(i, carry):
            for buf in (0, 1):
                c = i * 2 + buf
                _, cps = gather_copies(c, buf)
                for cp in cps:
                    cp.wait()
                @pl.when(c >= 1)
                def _():
                    out_copy(c - 1, 1 - buf).wait()

                @pl.when(c + 1 < chunks)
                def _():
                    start_chunk(c + 1, 1 - buf)

                add_pos(buf)
                out_copy(c, buf).start()
            return carry

        lax.fori_loop(0, chunks // 2, step, 0)
        out_copy(chunks - 1, 1).wait()

    return lookup


def kernel(indices, token_table, pos_table):
    B, S = indices.shape
    V, D = token_table.shape
    pos_rows = lax.slice(pos_table, (0, 0), (S, D))
    lookup = _make_lookup(B, S, D)
    return lookup(token_table, indices.astype(jnp.int32), pos_rows)
